# Initial kernel scaffold; baseline (speedup 1.0000x reference)
import jax
import jax.numpy as jnp
from jax.experimental import pallas as pl


def kernel(x, weight, bias_vec):
    raise NotImplementedError("write your pallas kernel here")


# speedup vs baseline: 6.7224x; 6.7224x over previous
import jax
import jax.numpy as jnp
from jax import lax
from jax.experimental import pallas as pl
from jax.experimental.pallas import tpu as pltpu


_DN = (((1,), (1,)), ((), ()))


def _dual_out_kernel(x_ref, w_ref, b_ref, y_out, yref_out):
    xb = x_ref[...].astype(jnp.bfloat16)
    wb = w_ref[...].astype(jnp.bfloat16)
    acc = lax.dot_general(xb, wb, dimension_numbers=_DN,
                          preferred_element_type=jnp.float32)
    b = b_ref[...]
    y_out[...] = (acc + b).astype(y_out.dtype)
    yref_out[...] = (0.5 * acc + b).astype(yref_out.dtype)


def _round_up(a, m):
    return ((a + m - 1) // m) * m


def kernel(x, weight, bias_vec):
    B, In = x.shape
    Out, In_w = weight.shape
    assert In_w == In

    bm = min(1024, _round_up(B, 8))
    bn = min(512, _round_up(Out, 128))

    Bp = _round_up(B, bm)
    Outp = _round_up(Out, bn)
    Inp = _round_up(In, 128)

    xp = x if (Bp, Inp) == (B, In) else jnp.pad(x, ((0, Bp - B), (0, Inp - In)))
    wp = (weight if (Outp, Inp) == (Out, In)
          else jnp.pad(weight, ((0, Outp - Out), (0, Inp - In))))
    bp = bias_vec if Outp == Out else jnp.pad(bias_vec, (0, Outp - Out))
    b2 = bp.reshape(1, Outp)

    grid = (Bp // bm, Outp // bn)

    y_p, yref_p = pl.pallas_call(
        _dual_out_kernel,
        out_shape=(jax.ShapeDtypeStruct((Bp, Outp), x.dtype),
                   jax.ShapeDtypeStruct((Bp, Outp), x.dtype)),
        grid=grid,
        in_specs=[pl.BlockSpec((bm, Inp), lambda i, j: (i, 0)),
                  pl.BlockSpec((bn, Inp), lambda i, j: (j, 0)),
                  pl.BlockSpec((1, bn), lambda i, j: (0, j))],
        out_specs=[pl.BlockSpec((bm, bn), lambda i, j: (i, j)),
                   pl.BlockSpec((bm, bn), lambda i, j: (i, j))],
        compiler_params=pltpu.CompilerParams(
            dimension_semantics=("parallel", "parallel"),
            vmem_limit_bytes=64 * 1024 * 1024),
    )(xp, wp, b2)

    if (Bp, Outp) == (B, Out):
        return y_p, yref_p
    return y_p[:B, :Out], yref_p[:B, :Out]
